# Initial kernel scaffold; baseline (speedup 1.0000x reference)
import jax
import jax.numpy as jnp
from jax.experimental import pallas as pl


def kernel(x, w1, b1, g1, be1, w2, b2, g2, be2, w3, b3):
    raise NotImplementedError("write your pallas kernel here")


# speedup vs baseline: 16.6475x; 16.6475x over previous
import functools

import jax
import jax.numpy as jnp
from jax import lax
from jax.experimental import pallas as pl
from jax.experimental.pallas import tpu as pltpu
from jax.experimental.pallas import tpu_sc as plsc

KK = 13
MW = 15
WN = 2 * MW + 1
EPS = 1e-5
SENT = 1e4
B, H, W = 4, 64, 64
C = 8
PW = W + 2 * MW
PH = H + 2 * MW
RB = 8
BIG = 3e38


def _conv2d(x, w, b):
    y = lax.conv_general_dilated(x, w, window_strides=(1, 1), padding='SAME',
                                 dimension_numbers=('NCHW', 'OIHW', 'NCHW'))
    return y + b[None, :, None, None]


def _batchnorm(x, g, b):
    m = jnp.mean(x, axis=(0, 2, 3), keepdims=True)
    v = jnp.var(x, axis=(0, 2, 3), keepdims=True)
    return g[None, :, None, None] * (x - m) / jnp.sqrt(v + EPS) \
        + b[None, :, None, None]


def _embed(x, w1, b1, g1, be1, w2, b2, g2, be2, w3, b3):
    h = jax.nn.relu(_batchnorm(_conv2d(x, w1, b1), g1, be1))
    h = jax.nn.relu(_batchnorm(_conv2d(h, w2, b2), g2, be2))
    return _conv2d(h, w3, b3)


def _knn_body(ep_ref, idx_ref):
    b = pl.program_id(0)
    band = pl.program_id(1)
    r0 = band * RB

    ec = ep_ref[0, :, pl.ds(r0 + MW, RB), MW:MW + W]
    rvec = lax.broadcasted_iota(jnp.int32, (RB, W), 0)
    wvec = lax.broadcasted_iota(jnp.int32, (RB, W), 1)
    fbase = (rvec + r0) * PW + wvec + b * (PH * PW)

    init_vals = tuple(jnp.full((RB, W), BIG, jnp.float32) for _ in range(KK))
    init_idxs = tuple(jnp.zeros((RB, W), jnp.int32) for _ in range(KK))

    def dy_body(dy, carry):
        vals, idxs = carry
        sw = ep_ref[0, :, pl.ds(r0 + dy, RB), :]
        for dx in range(WN):
            sh = lax.slice(sw, (0, 0, dx), (C, RB, dx + W))
            d = ec - sh
            d2 = jnp.sum(d * d, axis=0)
            if dx == MW:
                d2 = jnp.where(dy == MW, BIG, d2)
            cv = d2
            ci = fbase + (dy * PW + dx)
            new_vals, new_idxs = [], []
            for k in range(KK):
                vk, ik = vals[k], idxs[k]
                cond = cv < vk
                new_vals.append(jnp.where(cond, cv, vk))
                new_idxs.append(jnp.where(cond, ci, ik))
                cv = jnp.where(cond, vk, cv)
                ci = jnp.where(cond, ik, ci)
            vals, idxs = tuple(new_vals), tuple(new_idxs)
        return vals, idxs

    vals, idxs = lax.fori_loop(0, WN, dy_body, (init_vals, init_idxs))
    for k in range(KK):
        idx_ref[0, k, :, :] = idxs[k]


NROWS = B * KK * H * W
NW = 32
PER_W = NROWS // NW
CHUNK = 128
NCH = PER_W // CHUNK


@functools.cache
def _make_gather_sc():
    mesh = plsc.VectorSubcoreMesh(core_axis_name="c", subcore_axis_name="s")

    @functools.partial(
        pl.kernel,
        mesh=mesh,
        out_type=jax.ShapeDtypeStruct((NROWS // CHUNK, CHUNK, 16),
                                      jnp.float32),
        scratch_types=[
            pltpu.VMEM((NCH, CHUNK), jnp.int32),
            pltpu.VMEM((CHUNK, 16), jnp.float32),
            pltpu.SemaphoreType.DMA,
        ],
        compiler_params=pltpu.CompilerParams(use_tc_tiling_on_sc=False),
    )
    def _gather_sc(idx_hbm, tab_hbm, out_hbm, idx_v, rows_v, sem):
        wid = lax.axis_index("s") * 2 + lax.axis_index("c")
        base = wid * NCH
        pltpu.sync_copy(idx_hbm.at[wid], idx_v)

        def chunk_body(j, carry):
            pltpu.async_copy(tab_hbm.at[idx_v.at[j]], rows_v, sem).wait()
            pltpu.sync_copy(rows_v, out_hbm.at[base + j])
            return carry

        lax.fori_loop(0, NCH, chunk_body, 0)

    return _gather_sc


def kernel(x, w1, b1, g1, be1, w2, b2, g2, be2, w3, b3):
    e = _embed(x, w1, b1, g1, be1, w2, b2, g2, be2, w3, b3)
    ep = jnp.pad(e, ((0, 0), (0, 0), (MW, MW), (MW, MW)),
                 constant_values=SENT)

    idx = pl.pallas_call(
        _knn_body,
        grid=(B, H // RB),
        in_specs=[pl.BlockSpec((1, C, PH, PW), lambda b, r: (b, 0, 0, 0))],
        out_specs=pl.BlockSpec((1, KK, RB, W), lambda b, r: (b, 0, r, 0)),
        out_shape=jax.ShapeDtypeStruct((B, KK, H, W), jnp.int32),
    )(ep)

    tab = jnp.pad(jnp.transpose(ep, (0, 2, 3, 1)).reshape(B * PH * PW, C),
                  ((0, 0), (0, 16 - C)))
    idx2 = idx.reshape(NW, NCH, CHUNK)
    rows = _make_gather_sc()(idx2, tab)

    z = rows.reshape(B, KK, H, W, 16)[..., :C]
    zt = jnp.transpose(z, (0, 1, 4, 2, 3)).reshape(B, KK * C, H, W)
    return jnp.concatenate([e, zt], axis=1)
